# baseline (speedup 1.0000x reference)
import jax
import jax.numpy as jnp
from jax.experimental import pallas as pl
from jax.experimental.pallas import tpu as pltpu

USER_N = 6000
ITEM_N = 4000
NTOT = USER_N + ITEM_N
LAT = 32
HYP = 128
BM = 200
RBLKS = NTOT // BM
UBLKS = USER_N // BM
NB = 4
VLIM = 100 * 1024 * 1024

_F32 = jnp.float32


def _dotT(a, b):
    return jax.lax.dot_general(a, b, (((0,), (0,)), ((), ())),
                               preferred_element_type=_F32)


def _issue(adj_ref, abuf, sem, s):
    blk = jax.lax.rem(s, RBLKS)
    slot = jax.lax.rem(s, NB)
    pltpu.make_async_copy(
        adj_ref.at[pl.ds(blk * BM, BM), :], abuf.at[slot], sem.at[slot]
    ).start()


def _wait(adj_ref, abuf, sem, s):
    blk = jax.lax.rem(s, RBLKS)
    slot = jax.lax.rem(s, NB)
    pltpu.make_async_copy(
        adj_ref.at[pl.ds(blk * BM, BM), :], abuf.at[slot], sem.at[slot]
    ).wait()
    return abuf[slot]


def _body(adj_ref, emb_ref, embblk_ref, uH_ref, iH_ref,
          tem_ref, h_ref, eo_ref,
          abuf, sem, rhs_s, e1_s, Ku_s, Ki_s, Pu_s, Pi_s, Lu_s, Li_s):
    r = pl.program_id(0)
    rm = jax.lax.rem(r, RBLKS)

    @pl.when(r == 0)
    def _prologue():
        for j in range(NB - 1):
            _issue(adj_ref, abuf, sem, j)
        rhs_s[...] = emb_ref[...]
        uE = emb_ref[:USER_N, :]
        iE = emb_ref[USER_N:, :]
        Gu = _dotT(uE, uE)
        Gi = _dotT(iE, iE)
        Ku_s[...] = jnp.dot(uH_ref[...], _dotT(uH_ref[...], Gu),
                            preferred_element_type=_F32)
        Ki_s[...] = jnp.dot(iH_ref[...], _dotT(iH_ref[...], Gi),
                            preferred_element_type=_F32)
        Pu_s[...] = jnp.zeros_like(Pu_s)
        Pi_s[...] = jnp.zeros_like(Pi_s)

    @pl.when(r + NB - 1 < 2 * RBLKS)
    def _prefetch():
        _issue(adj_ref, abuf, sem, r + NB - 1)

    @pl.when(r == RBLKS)
    def _swap_rhs():
        rhs_s[...] = e1_s[...]

    a = _wait(adj_ref, abuf, sem, r)
    tem = jnp.dot(a, rhs_s[...], preferred_element_type=_F32)
    eblk = embblk_ref[...]
    K = jnp.where(r < RBLKS,
                  jnp.where(rm < UBLKS, Ku_s[...], Ki_s[...]),
                  jnp.where(rm < UBLKS, Lu_s[...], Li_s[...]))
    h = jnp.dot(eblk, K, preferred_element_type=_F32)
    lat = tem + h
    tem_ref[...] = tem.reshape(1, BM, LAT)
    h_ref[...] = h.reshape(1, BM, LAT)

    @pl.when(r < RBLKS)
    def _pass1_epi():
        e1_s[pl.ds(rm * BM, BM), :] = lat
        eo_ref[...] = lat.reshape(1, BM, LAT)
        contrib = _dotT(eblk, lat)

        @pl.when(rm < UBLKS)
        def _():
            Pu_s[...] += contrib

        @pl.when(rm >= UBLKS)
        def _():
            Pi_s[...] += contrib

    @pl.when(r == RBLKS - 1)
    def _mid():
        Lu_s[...] = jnp.dot(uH_ref[...], _dotT(uH_ref[...], Pu_s[...]),
                            preferred_element_type=_F32)
        Li_s[...] = jnp.dot(iH_ref[...], _dotT(iH_ref[...], Pi_s[...]),
                            preferred_element_type=_F32)

    @pl.when(r >= RBLKS)
    def _pass2_epi():
        out = eblk + e1_s[pl.ds(rm * BM, BM), :] + lat
        eo_ref[...] = out.reshape(1, BM, LAT)


def _stk_spec():
    return pl.BlockSpec((1, BM, LAT),
                        lambda r: (r // RBLKS, jax.lax.rem(r, RBLKS), 0))


_call = pl.pallas_call(
    _body,
    grid=(2 * RBLKS,),
    in_specs=[
        pl.BlockSpec(memory_space=pl.ANY),
        pl.BlockSpec((NTOT, LAT), lambda r: (0, 0)),
        pl.BlockSpec((BM, LAT),
                     lambda r: (jax.lax.rem(r, RBLKS), 0)),
        pl.BlockSpec((LAT, HYP), lambda r: (0, 0)),
        pl.BlockSpec((LAT, HYP), lambda r: (0, 0)),
    ],
    out_specs=[_stk_spec(), _stk_spec(), _stk_spec()],
    out_shape=[
        jax.ShapeDtypeStruct((2, NTOT, LAT), _F32),
        jax.ShapeDtypeStruct((2, NTOT, LAT), _F32),
        jax.ShapeDtypeStruct((2, NTOT, LAT), _F32),
    ],
    scratch_shapes=[
        pltpu.VMEM((NB, BM, NTOT), _F32),
        pltpu.SemaphoreType.DMA((NB,)),
        pltpu.VMEM((NTOT, LAT), _F32),
        pltpu.VMEM((NTOT, LAT), _F32),
        pltpu.VMEM((LAT, LAT), _F32),
        pltpu.VMEM((LAT, LAT), _F32),
        pltpu.VMEM((LAT, LAT), _F32),
        pltpu.VMEM((LAT, LAT), _F32),
        pltpu.VMEM((LAT, LAT), _F32),
        pltpu.VMEM((LAT, LAT), _F32),
    ],
    compiler_params=pltpu.CompilerParams(
        dimension_semantics=("arbitrary",),
        vmem_limit_bytes=VLIM),
)


def kernel(adj, keepRate, uEmbeds, iEmbeds, uHyper, iHyper):
    del keepRate
    emb = jnp.concatenate([uEmbeds, iEmbeds], axis=0)
    tem, h, eo = _call(adj, emb, emb, uHyper, iHyper)
    return (eo[1], tem[0], tem[1], h[0], h[1])
